# Initial kernel scaffold; baseline (speedup 1.0000x reference)
import jax
import jax.numpy as jnp
from jax.experimental import pallas as pl


def kernel(indices, table):
    raise NotImplementedError("write your pallas kernel here")


# speedup vs baseline: 1.1034x; 1.1034x over previous
import functools

import jax
import jax.numpy as jnp
from jax import lax
from jax.experimental import pallas as pl
from jax.experimental.pallas import tpu as pltpu
from jax.experimental.pallas import tpu_sc as plsc

EMBED = 64

_info = plsc.get_sparse_core_info()
_NC = _info.num_cores
_NS = _info.num_subcores
_NW = _NC * _NS

_CHUNK = 128


def _emb_body(n_chunks, table_hbm, idx_hbm, out_hbm, idx_v, buf, gsem):
    wid = lax.axis_index("s") * _NC + lax.axis_index("c")
    base = wid * (n_chunks * _CHUNK)
    pltpu.sync_copy(idx_hbm.at[wid], idx_v)
    for c in range(n_chunks):
        pltpu.async_copy(table_hbm.at[idx_v.at[c]], buf, gsem).wait()
        pltpu.sync_copy(buf, out_hbm.at[pl.ds(base + c * _CHUNK, _CHUNK)])


def kernel(indices, table):
    batch, hist = indices.shape
    n_total = batch * hist
    assert n_total % (_NW * _CHUNK) == 0
    n_chunks = n_total // (_NW * _CHUNK)
    idx3 = indices.reshape(_NW, n_chunks, _CHUNK)

    k = pl.kernel(
        functools.partial(_emb_body, n_chunks),
        out_type=jax.ShapeDtypeStruct((n_total, EMBED), jnp.float32),
        mesh=plsc.VectorSubcoreMesh(core_axis_name="c", subcore_axis_name="s"),
        scratch_types=[
            pltpu.VMEM((n_chunks, _CHUNK), jnp.int32),
            pltpu.VMEM((_CHUNK, EMBED), jnp.float32),
            pltpu.SemaphoreType.DMA,
        ],
        compiler_params=pltpu.CompilerParams(use_tc_tiling_on_sc=False),
    )
    out = k(table, idx3)
    return out.reshape(batch, hist, EMBED)


# speedup vs baseline: 1.1570x; 1.0486x over previous
import functools

import jax
import jax.numpy as jnp
from jax import lax
from jax.experimental import pallas as pl
from jax.experimental.pallas import tpu as pltpu
from jax.experimental.pallas import tpu_sc as plsc

EMBED = 64

_info = plsc.get_sparse_core_info()
_NC = _info.num_cores
_NS = _info.num_subcores
_NW = _NC * _NS

_CHUNK = 128
_NBUF = 5


def _emb_body(n_chunks, table_hbm, idx_hbm, out_hbm, *scratch):
    idx_v = scratch[0]
    bufs = scratch[1:1 + _NBUF]
    gsems = scratch[1 + _NBUF:1 + 2 * _NBUF]
    osems = scratch[1 + 2 * _NBUF:1 + 3 * _NBUF]

    wid = lax.axis_index("s") * _NC + lax.axis_index("c")
    base = wid * (n_chunks * _CHUNK)
    pltpu.sync_copy(idx_hbm.at[wid], idx_v)

    def issue_gather(c, b):
        pltpu.async_copy(table_hbm.at[idx_v.at[c]], bufs[b], gsems[b])

    def wait_gather(b):
        pltpu.make_async_copy(table_hbm.at[idx_v.at[0]], bufs[b], gsems[b]).wait()

    def issue_out(c, b):
        pltpu.async_copy(bufs[b], out_hbm.at[pl.ds(base + c * _CHUNK, _CHUNK)],
                         osems[b])

    def wait_out(b):
        pltpu.make_async_copy(bufs[b], out_hbm.at[pl.ds(base, _CHUNK)],
                              osems[b]).wait()

    for b in range(_NBUF):
        issue_gather(b, b)

    n_outer = n_chunks // _NBUF

    def outer(g, carry):
        for b in range(_NBUF):
            c = g * _NBUF + b
            wait_gather(b)
            issue_out(c, b)
            wait_out(b)
            issue_gather(c + _NBUF, b)
        return carry

    lax.fori_loop(0, n_outer - 1, outer, 0)

    for b in range(_NBUF):
        c = (n_outer - 1) * _NBUF + b
        wait_gather(b)
        issue_out(c, b)
    for b in range(_NBUF):
        wait_out(b)


def kernel(indices, table):
    batch, hist = indices.shape
    n_total = batch * hist
    assert n_total % (_NW * _CHUNK) == 0
    n_chunks = n_total // (_NW * _CHUNK)
    assert n_chunks % _NBUF == 0
    idx3 = indices.reshape(_NW, n_chunks, _CHUNK)

    k = pl.kernel(
        functools.partial(_emb_body, n_chunks),
        out_type=jax.ShapeDtypeStruct((n_total, EMBED), jnp.float32),
        mesh=plsc.VectorSubcoreMesh(core_axis_name="c", subcore_axis_name="s"),
        scratch_types=(
            [pltpu.VMEM((n_chunks, _CHUNK), jnp.int32)]
            + [pltpu.VMEM((_CHUNK, EMBED), jnp.float32) for _ in range(_NBUF)]
            + [pltpu.SemaphoreType.DMA for _ in range(2 * _NBUF)]
        ),
        compiler_params=pltpu.CompilerParams(use_tc_tiling_on_sc=False),
    )
    out = k(table, idx3)
    return out.reshape(batch, hist, EMBED)


# speedup vs baseline: 2.2465x; 1.9416x over previous
import functools

import jax
import jax.numpy as jnp
from jax import lax
from jax.experimental import pallas as pl
from jax.experimental.pallas import tpu as pltpu
from jax.experimental.pallas import tpu_sc as plsc

EMBED = 64

_info = plsc.get_sparse_core_info()
_NC = _info.num_cores
_NS = _info.num_subcores
_NW = _NC * _NS


def _emb_body(bpw, hist, table3, idx3, out3, idx_v, ob0, ob1, ob2, ob3,
              gs0, gs1, gs2, gs3, os0, os1, os2, os3):
    wid = lax.axis_index("s") * _NC + lax.axis_index("c")
    obufs = (ob0, ob1, ob2, ob3)
    gsems = (gs0, gs1, gs2, gs3)
    osems = (os0, os1, os2, os3)

    pltpu.sync_copy(idx3.at[wid], idx_v)

    def issue_gathers(j, pp):
        s = 0
        while s < hist:
            base = min(s, hist - 16)
            v = idx_v[j, pl.ds(base, 16)]
            for lane in range(s - base, min(hist, base + 16) - base):
                e = v[lane]
                pltpu.async_copy(table3.at[e >> 3, e & 7],
                                 obufs[pp].at[base + lane], gsems[pp])
            s = base + 16

    def wait_gathers(pp):
        pltpu.make_async_copy(out3.at[0], obufs[pp], gsems[pp]).wait()

    def issue_out(j, pp):
        pltpu.async_copy(obufs[pp], out3.at[wid * bpw + j], osems[pp])

    def wait_out(pp):
        pltpu.make_async_copy(obufs[pp], out3.at[0], osems[pp]).wait()

    for g in range(4):
        issue_gathers(g, g)

    def outer(j2, carry):
        for pp in range(4):
            j = j2 * 4 + pp
            wait_gathers(pp)
            issue_out(j, pp)
            qq = (pp + 2) % 4

            @pl.when((j >= 2) & (j + 2 < bpw))
            def _():
                wait_out(qq)
                issue_gathers(j + 2, qq)
        return carry

    lax.fori_loop(0, bpw // 4, outer, 0)

    for pp in range(4):
        wait_out(pp)


def kernel(indices, table):
    batch, hist = indices.shape
    vocab = table.shape[0]
    assert batch % _NW == 0 and vocab % 8 == 0
    bpw = batch // _NW
    assert bpw % 4 == 0 and bpw >= 8
    table3 = table.reshape(vocab // 8, 8, EMBED)
    idx3 = indices.reshape(_NW, bpw, hist)

    k = pl.kernel(
        functools.partial(_emb_body, bpw, hist),
        out_type=jax.ShapeDtypeStruct((batch, hist, EMBED), jnp.float32),
        mesh=plsc.VectorSubcoreMesh(core_axis_name="c", subcore_axis_name="s"),
        scratch_types=(
            [pltpu.VMEM((bpw, hist), jnp.int32)]
            + [pltpu.VMEM((hist, EMBED), jnp.float32) for _ in range(4)]
            + [pltpu.SemaphoreType.DMA for _ in range(8)]
        ),
    )
    return k(table3, idx3)
